# baseline (speedup 1.0000x reference)
import functools

import jax
import jax.numpy as jnp
from jax import lax
from jax.experimental import pallas as pl
from jax.experimental.pallas import tpu as pltpu
from jax.experimental.pallas import tpu_sc as plsc

D = 64
NC, NS = 2, 16
NW = NC * NS
GROUP = 128
K = 2
CHUNK = K * GROUP
NBUF = 5
TOTAL = 16384 * 50
PER_W = TOTAL // NW
N_GROUPS = PER_W // GROUP
N_CHUNKS = PER_W // CHUNK
N_ITERS = N_CHUNKS // NBUF

_mesh = plsc.VectorSubcoreMesh(core_axis_name="c", subcore_axis_name="s")


@functools.partial(
    pl.kernel,
    out_type=jax.ShapeDtypeStruct((TOTAL, D), jnp.float32),
    mesh=_mesh,
    scratch_types=[
        pltpu.VMEM((N_GROUPS, GROUP), jnp.int32),
        *[pltpu.VMEM((CHUNK, D), jnp.float32) for _ in range(NBUF)],
        *[pltpu.SemaphoreType.DMA for _ in range(NBUF)],
        *[pltpu.SemaphoreType.DMA for _ in range(NBUF)],
    ],
    compiler_params=pltpu.CompilerParams(use_tc_tiling_on_sc=False),
)
def _sc_gather(words_hbm, table_hbm, out_hbm, idx_v, *bufs_and_sems):
    rows = bufs_and_sems[:NBUF]
    gsem = bufs_and_sems[NBUF:2 * NBUF]
    osem = bufs_and_sems[2 * NBUF:]

    wid = lax.axis_index("s") * NC + lax.axis_index("c")
    base = wid * PER_W
    pltpu.sync_copy(words_hbm.at[pl.ds(wid * N_GROUPS, N_GROUPS)], idx_v)

    def fire_gathers(chunk, b):
        for j in range(K):
            pltpu.async_copy(
                table_hbm.at[idx_v.at[chunk * K + j]],
                rows[b].at[pl.ds(j * GROUP, GROUP)],
                gsem[b],
            )

    def wait_chunk_gathers(b):
        pltpu.make_async_copy(
            table_hbm.at[pl.ds(0, CHUNK)], rows[b], gsem[b]
        ).wait()

    def wait_out(b):
        pltpu.make_async_copy(
            rows[b], out_hbm.at[pl.ds(base, CHUNK)], osem[b]
        ).wait()

    for b in range(NBUF):
        fire_gathers(b, b)

    def slot_group(i, _):
        for b in range(NBUF):
            g = i * NBUF + b
            wait_chunk_gathers(b)
            pltpu.async_copy(
                rows[b], out_hbm.at[pl.ds(base + g * CHUNK, CHUNK)], osem[b]
            )
            t = g + NBUF - 2
            bt = (b + NBUF - 2) % NBUF

            @pl.when(jnp.logical_and(t >= NBUF, t < N_CHUNKS))
            def _refill():
                wait_out(bt)
                fire_gathers(t, bt)
        return ()

    lax.fori_loop(0, N_ITERS, slot_group, (), unroll=False)

    for b in range(NBUF):
        wait_out(b)


def kernel(words, table):
    flat = words.reshape(TOTAL // GROUP, GROUP)
    out = _sc_gather(flat, table)
    return out.reshape(words.shape[0], words.shape[1], D)
